# baseline (speedup 1.0000x reference)
import functools

import jax
import jax.numpy as jnp
from jax import lax
from jax.experimental import pallas as pl
from jax.experimental.pallas import tpu as pltpu
from jax.experimental.pallas import tpu_sc as plsc

C = 128
O = 128
F_FULL = 50000
F = 25000
K = 16
NK = K + 1

BF_A = 512
F_FULL_PAD = 98 * BF_A

NW = 32
FACES_PER_W = 784
F_PAD = NW * FACES_PER_W
FACES_PER_GATHER = 7
ROWS_PER_GATHER = 128
GATHERS_PER_W = FACES_PER_W // FACES_PER_GATHER
NBUF = 4
NITER = GATHERS_PER_W // NBUF

BF_C = 512
NBLK_C = F_PAD // BF_C


def _matmul_body(fea_ref, w_ref, out_ref):
    i = pl.program_id(0)
    y = lax.dot_general(
        fea_ref[...], w_ref[...],
        dimension_numbers=(((0,), (1,)), ((), ())),
        preferred_element_type=jnp.float32,
    )
    rows = lax.broadcasted_iota(jnp.int32, (BF_A, O), 0) + i * BF_A
    out_ref[...] = jnp.where(rows < F_FULL, y, 0.0).astype(jnp.bfloat16)


def _matmul_transposed(fea2d, W):
    return pl.pallas_call(
        _matmul_body,
        grid=(F_FULL_PAD // BF_A,),
        in_specs=[
            pl.BlockSpec((C, BF_A), lambda i: (0, i)),
            pl.BlockSpec((O, C), lambda i: (0, 0)),
        ],
        out_specs=pl.BlockSpec((BF_A, O), lambda i: (i, 0)),
        out_shape=jax.ShapeDtypeStruct((F_FULL_PAD, O), jnp.bfloat16),
    )(fea2d, W)


def _sc_gather_sum(gT, ring_flat, pool_pad):
    mesh = plsc.VectorSubcoreMesh(core_axis_name="c", subcore_axis_name="s")

    @functools.partial(
        pl.kernel,
        mesh=mesh,
        out_type=jax.ShapeDtypeStruct((F_PAD, O), jnp.bfloat16),
        compiler_params=pltpu.CompilerParams(use_tc_tiling_on_sc=False,
                                             needs_layout_passes=False),
        scratch_types=[
            pltpu.VMEM((GATHERS_PER_W, ROWS_PER_GATHER), jnp.int32),
            pltpu.VMEM((FACES_PER_W * K,), jnp.int32),
            pltpu.VMEM((FACES_PER_W,), jnp.int32),
            *[pltpu.VMEM((ROWS_PER_GATHER, O), jnp.bfloat16)
              for _ in range(NBUF)],
            *[pltpu.VMEM((FACES_PER_GATHER, O), jnp.bfloat16)
              for _ in range(NBUF)],
            *[pltpu.SemaphoreType.DMA for _ in range(2 * NBUF)],
        ],
    )
    def k(gT_hbm, ring_hbm, pool_hbm, out_hbm, idx_v, rbuf, pbuf,
          b0, b1, b2, b3, st0, st1, st2, st3,
          s0, s1, s2, s3, t0, t1, t2, t3):
        bufs = (b0, b1, b2, b3)
        stag = (st0, st1, st2, st3)
        sems = (s0, s1, s2, s3)
        osems = (t0, t1, t2, t3)
        wid = lax.axis_index("s") * 2 + lax.axis_index("c")
        row_base = wid * FACES_PER_W

        pltpu.sync_copy(ring_hbm.at[pl.ds(row_base * K, FACES_PER_W * K)],
                        rbuf)
        pltpu.sync_copy(pool_hbm.at[pl.ds(row_base, FACES_PER_W)], pbuf)
        lane = lax.iota(jnp.int32, 16)
        in7 = lane < FACES_PER_GATHER

        def build(g, _):
            f0 = g * FACES_PER_GATHER
            for j in range(FACES_PER_GATHER):
                idx_v[g, pl.ds(j * K, K)] = rbuf[pl.ds((f0 + j) * K, K)]
            pool = plsc.load_gather(pbuf, [jnp.where(in7, f0 + lane, 0)])
            idx_v[g, pl.ds(FACES_PER_GATHER * K, 16)] = (
                jnp.where(in7, pool, F_FULL))
            return 0

        lax.fori_loop(0, GATHERS_PER_W, build, 0)

        def issue(g, u):
            return pltpu.async_copy(gT_hbm.at[idx_v.at[g]], bufs[u], sems[u])

        def out_slice(g):
            return out_hbm.at[pl.ds(row_base + g * FACES_PER_GATHER,
                                    FACES_PER_GATHER)]

        for u in range(NBUF):
            issue(u, u)

        def body(t, _):
            for u in range(NBUF):
                g = t * NBUF + u
                pltpu.make_async_copy(gT_hbm.at[idx_v.at[g]],
                                      bufs[u], sems[u]).wait()
                for j in range(FACES_PER_GATHER):
                    for c in range(O // 32):
                        sl = pl.ds(c * 32, 32)
                        acc = bufs[u][FACES_PER_GATHER * K + j, sl]
                        for r in range(K):
                            acc = acc + bufs[u][j * K + r, sl]
                        stag[u][j, sl] = acc

                @pl.when(t < NITER - 1)
                def _():
                    issue(g + NBUF, u)

                @pl.when(t > 0)
                def _():
                    pltpu.make_async_copy(stag[u], out_slice(0),
                                          osems[u]).wait()

                pltpu.async_copy(stag[u], out_slice(g), osems[u])
            return 0

        lax.fori_loop(0, NITER, body, 0)
        for u in range(NBUF):
            pltpu.make_async_copy(stag[u], out_slice(0), osems[u]).wait()

    return k(gT, ring_flat, pool_pad)


def _bn_body(yT_ref, gb_ref, out_ref, acc_ref):
    i = pl.program_id(0)

    @pl.when(i == 0)
    def _():
        acc_ref[...] = jnp.zeros_like(acc_ref)

    @pl.when(i < NBLK_C)
    def _():
        rows = lax.broadcasted_iota(jnp.int32, (BF_C, O), 0) + i * BF_C
        y = jnp.where(rows < F, yT_ref[...].astype(jnp.float32), 0.0)
        acc_ref[0:1, :] += jnp.sum(y, axis=0, keepdims=True)
        acc_ref[1:2, :] += jnp.sum(y * y, axis=0, keepdims=True)

    @pl.when(i >= NBLK_C)
    def _():
        mean = acc_ref[0:1, :] / F
        var = acc_ref[1:2, :] / F - mean * mean
        inv = lax.rsqrt(var + 1e-5)
        scale = gb_ref[0:1, :] * inv
        shift = gb_ref[1:2, :] - mean * scale
        y = yT_ref[...].astype(jnp.float32)
        z = jnp.maximum(y * scale + shift, 0.0)
        eye = (lax.broadcasted_iota(jnp.int32, (O, O), 0)
               == lax.broadcasted_iota(jnp.int32, (O, O), 1)
               ).astype(jnp.float32)
        out_ref[...] = lax.dot_general(
            eye, z,
            dimension_numbers=(((1,), (1,)), ((), ())),
            preferred_element_type=jnp.float32,
        )


def _bn_norm(yT, gb):
    return pl.pallas_call(
        _bn_body,
        grid=(2 * NBLK_C,),
        in_specs=[
            pl.BlockSpec((BF_C, O), lambda i: (i % NBLK_C, 0)),
            pl.BlockSpec((2, O), lambda i: (0, 0)),
        ],
        out_specs=pl.BlockSpec((O, BF_C),
                               lambda i: (0, jnp.maximum(i - NBLK_C, 0))),
        out_shape=jax.ShapeDtypeStruct((O, F), jnp.float32),
        scratch_shapes=[pltpu.VMEM((2, O), jnp.float32)],
    )(yT, gb)


def kernel(fea, ring_n, pool_idx, W, b, gamma, beta):
    del b
    fea2d = fea[0]
    gT = _matmul_transposed(fea2d, W)

    ring_flat = jnp.pad(ring_n.reshape(-1), (0, (F_PAD - F) * K),
                        constant_values=F_FULL)
    pool_pad = jnp.pad(pool_idx, (0, F_PAD - F), constant_values=F_FULL)

    yT = _sc_gather_sum(gT, ring_flat, pool_pad)
    gb = jnp.stack([gamma, beta])
    out2d = _bn_norm(yT, gb)
    return out2d[None]


# speedup vs baseline: 2.6171x; 2.6171x over previous
import functools

import jax
import jax.numpy as jnp
from jax import lax
from jax.experimental import pallas as pl
from jax.experimental.pallas import tpu as pltpu
from jax.experimental.pallas import tpu_sc as plsc

C = 128
O = 128
F_FULL = 50000
F = 25000
K = 16
NK = K + 1

BF_A = 512
F_FULL_PAD = 98 * BF_A

NW = 32
FACES_PER_W = 784
F_PAD = NW * FACES_PER_W
FACES_PER_GATHER = 4
ROWS_PER_GATHER = FACES_PER_GATHER * NK
GATHERS_PER_W = FACES_PER_W // FACES_PER_GATHER
NBUF = 4
NITER = GATHERS_PER_W // NBUF

BF_C = 512
NBLK_C = F_PAD // BF_C


def _matmul_body(fea_ref, w_ref, out_ref):
    i = pl.program_id(0)
    y = lax.dot_general(
        fea_ref[...], w_ref[...],
        dimension_numbers=(((0,), (1,)), ((), ())),
        preferred_element_type=jnp.float32,
    )
    rows = lax.broadcasted_iota(jnp.int32, (BF_A, O), 0) + i * BF_A
    out_ref[...] = jnp.where(rows < F_FULL, y, 0.0).astype(jnp.bfloat16)


def _matmul_transposed(fea2d, W):
    return pl.pallas_call(
        _matmul_body,
        grid=(F_FULL_PAD // BF_A,),
        in_specs=[
            pl.BlockSpec((C, BF_A), lambda i: (0, i)),
            pl.BlockSpec((O, C), lambda i: (0, 0)),
        ],
        out_specs=pl.BlockSpec((BF_A, O), lambda i: (i, 0)),
        out_shape=jax.ShapeDtypeStruct((F_FULL_PAD, O), jnp.bfloat16),
    )(fea2d, W)


def _sc_gather_sum(gT, idx3d):
    mesh = plsc.VectorSubcoreMesh(core_axis_name="c", subcore_axis_name="s")

    @functools.partial(
        pl.kernel,
        mesh=mesh,
        out_type=jax.ShapeDtypeStruct((F_PAD, O), jnp.bfloat16),
        compiler_params=pltpu.CompilerParams(use_tc_tiling_on_sc=False),
        scratch_types=[
            pltpu.VMEM((GATHERS_PER_W, ROWS_PER_GATHER), jnp.int32),
            *[pltpu.VMEM((ROWS_PER_GATHER, O), jnp.bfloat16)
              for _ in range(NBUF)],
            *[pltpu.VMEM((2 * FACES_PER_GATHER, O), jnp.bfloat16)
              for _ in range(2)],
            *[pltpu.SemaphoreType.DMA for _ in range(NBUF + 2)],
        ],
    )
    def k(gT_hbm, idx_hbm, out_hbm, idx_v, b0, b1, b2, b3,
          st0, st1, s0, s1, s2, s3, t0, t1):
        bufs = (b0, b1, b2, b3)
        stag = (st0, st1)
        sems = (s0, s1, s2, s3)
        osems = (t0, t1)
        wid = lax.axis_index("s") * 2 + lax.axis_index("c")
        pltpu.sync_copy(idx_hbm.at[wid], idx_v)
        row_base = wid * FACES_PER_W

        def issue(g, u):
            return pltpu.async_copy(gT_hbm.at[idx_v.at[g]], bufs[u], sems[u])

        def out_slice(g):
            return out_hbm.at[pl.ds(row_base + g * FACES_PER_GATHER,
                                    2 * FACES_PER_GATHER)]

        for u in range(NBUF):
            issue(u, u)

        def body(t, _):
            for v in range(2):
                for h in range(2):
                    u = 2 * v + h
                    g = t * NBUF + u
                    pltpu.make_async_copy(gT_hbm.at[idx_v.at[g]],
                                          bufs[u], sems[u]).wait()
                    for j in range(FACES_PER_GATHER):
                        for c in range(O // 32):
                            sl = pl.ds(c * 32, 32)
                            acc = bufs[u][j * NK, sl]
                            for r in range(1, NK):
                                acc = acc + bufs[u][j * NK + r, sl]
                            stag[v][h * FACES_PER_GATHER + j, sl] = acc

                    @pl.when(t < NITER - 1)
                    def _():
                        issue(g + NBUF, u)

                @pl.when(t > 0)
                def _():
                    pltpu.make_async_copy(stag[v], out_slice(2 * v),
                                          osems[v]).wait()

                pltpu.async_copy(stag[v], out_slice(t * NBUF + 2 * v),
                                 osems[v])
            return 0

        lax.fori_loop(0, NITER, body, 0)
        for v in range(2):
            pltpu.make_async_copy(stag[v], out_slice(2 * v), osems[v]).wait()

    return k(gT, idx3d)


def _bn_body(yT_ref, gb_ref, out_ref, acc_ref):
    i = pl.program_id(0)

    @pl.when(i == 0)
    def _():
        acc_ref[...] = jnp.zeros_like(acc_ref)

    @pl.when(i < NBLK_C)
    def _():
        rows = lax.broadcasted_iota(jnp.int32, (BF_C, O), 0) + i * BF_C
        y = jnp.where(rows < F, yT_ref[...].astype(jnp.float32), 0.0)
        acc_ref[0:1, :] += jnp.sum(y, axis=0, keepdims=True)
        acc_ref[1:2, :] += jnp.sum(y * y, axis=0, keepdims=True)

    @pl.when(i >= NBLK_C)
    def _():
        mean = acc_ref[0:1, :] / F
        var = acc_ref[1:2, :] / F - mean * mean
        inv = lax.rsqrt(var + 1e-5)
        scale = gb_ref[0:1, :] * inv
        shift = gb_ref[1:2, :] - mean * scale
        y = yT_ref[...].astype(jnp.float32)
        z = jnp.maximum(y * scale + shift, 0.0)
        eye = (lax.broadcasted_iota(jnp.int32, (O, O), 0)
               == lax.broadcasted_iota(jnp.int32, (O, O), 1)
               ).astype(jnp.float32)
        out_ref[...] = lax.dot_general(
            eye, z,
            dimension_numbers=(((1,), (1,)), ((), ())),
            preferred_element_type=jnp.float32,
        )


def _bn_norm(yT, gb):
    return pl.pallas_call(
        _bn_body,
        grid=(2 * NBLK_C,),
        in_specs=[
            pl.BlockSpec((BF_C, O), lambda i: (i % NBLK_C, 0)),
            pl.BlockSpec((2, O), lambda i: (0, 0)),
        ],
        out_specs=pl.BlockSpec((O, BF_C),
                               lambda i: (0, jnp.maximum(i - NBLK_C, 0))),
        out_shape=jax.ShapeDtypeStruct((O, F), jnp.float32),
        scratch_shapes=[pltpu.VMEM((2, O), jnp.float32)],
    )(yT, gb)


def kernel(fea, ring_n, pool_idx, W, b, gamma, beta):
    del b
    fea2d = fea[0]
    gT = _matmul_transposed(fea2d, W)

    idx = jnp.concatenate([pool_idx[:, None], ring_n[0]], axis=1)
    idx = jnp.pad(idx, ((0, F_PAD - F), (0, 0)), constant_values=F_FULL)
    idx3d = idx.reshape(NW, GATHERS_PER_W, ROWS_PER_GATHER)

    yT = _sc_gather_sum(gT, idx3d)
    gb = jnp.stack([gamma, beta])
    out2d = _bn_norm(yT, gb)
    return out2d[None]


# speedup vs baseline: 2.8128x; 1.0748x over previous
import functools

import jax
import jax.numpy as jnp
from jax import lax
from jax.experimental import pallas as pl
from jax.experimental.pallas import tpu as pltpu
from jax.experimental.pallas import tpu_sc as plsc

C = 128
O = 128
F_FULL = 50000
F = 25000
K = 16
NK = K + 1

BF_A = 1024
F_FULL_PAD = 49 * BF_A

NW = 32
FACES_PER_W = 784
F_PAD = NW * FACES_PER_W
FACES_PER_GATHER = 4
ROWS_PER_GATHER = FACES_PER_GATHER * NK
GATHERS_PER_W = FACES_PER_W // FACES_PER_GATHER
NBUF = 4
NITER = GATHERS_PER_W // NBUF

BF_C = 512
NBLK_C = F_PAD // BF_C


def _matmul_body(fea_ref, w_ref, out_ref):
    i = pl.program_id(0)
    y = lax.dot_general(
        fea_ref[...], w_ref[...],
        dimension_numbers=(((0,), (1,)), ((), ())),
        preferred_element_type=jnp.float32,
    )
    rows = lax.broadcasted_iota(jnp.int32, (BF_A, O), 0) + i * BF_A
    out_ref[...] = jnp.where(rows < F_FULL, y, 0.0).astype(jnp.bfloat16)


def _matmul_transposed(fea2d, W):
    return pl.pallas_call(
        _matmul_body,
        grid=(F_FULL_PAD // BF_A,),
        in_specs=[
            pl.BlockSpec((C, BF_A), lambda i: (0, i)),
            pl.BlockSpec((O, C), lambda i: (0, 0)),
        ],
        out_specs=pl.BlockSpec((BF_A, O), lambda i: (i, 0)),
        out_shape=jax.ShapeDtypeStruct((F_FULL_PAD, O), jnp.bfloat16),
    )(fea2d, W)


def _sc_gather_sum(gT, idx3d):
    mesh = plsc.VectorSubcoreMesh(core_axis_name="c", subcore_axis_name="s")

    @functools.partial(
        pl.kernel,
        mesh=mesh,
        out_type=jax.ShapeDtypeStruct((F_PAD, O), jnp.bfloat16),
        compiler_params=pltpu.CompilerParams(use_tc_tiling_on_sc=False),
        scratch_types=[
            pltpu.VMEM((GATHERS_PER_W, ROWS_PER_GATHER), jnp.int32),
            *[pltpu.VMEM((ROWS_PER_GATHER, O), jnp.bfloat16)
              for _ in range(NBUF)],
            *[pltpu.VMEM((2 * FACES_PER_GATHER, O), jnp.bfloat16)
              for _ in range(2)],
            *[pltpu.SemaphoreType.DMA for _ in range(NBUF + 2)],
        ],
    )
    def k(gT_hbm, idx_hbm, out_hbm, idx_v, b0, b1, b2, b3,
          st0, st1, s0, s1, s2, s3, t0, t1):
        bufs = (b0, b1, b2, b3)
        stag = (st0, st1)
        sems = (s0, s1, s2, s3)
        osems = (t0, t1)
        wid = lax.axis_index("s") * 2 + lax.axis_index("c")
        pltpu.sync_copy(idx_hbm.at[wid], idx_v)
        row_base = wid * FACES_PER_W

        def issue(g, u):
            return pltpu.async_copy(gT_hbm.at[idx_v.at[g]], bufs[u], sems[u])

        def out_slice(g):
            return out_hbm.at[pl.ds(row_base + g * FACES_PER_GATHER,
                                    2 * FACES_PER_GATHER)]

        for u in range(NBUF):
            issue(u, u)

        def body(t, _):
            for v in range(2):
                for h in range(2):
                    u = 2 * v + h
                    g = t * NBUF + u
                    pltpu.make_async_copy(gT_hbm.at[idx_v.at[g]],
                                          bufs[u], sems[u]).wait()
                    for j in range(FACES_PER_GATHER):
                        for c in range(O // 32):
                            sl = pl.ds(c * 32, 32)
                            acc = bufs[u][j * NK, sl]
                            for r in range(1, NK):
                                acc = acc + bufs[u][j * NK + r, sl]
                            stag[v][h * FACES_PER_GATHER + j, sl] = acc

                    @pl.when(t < NITER - 1)
                    def _():
                        issue(g + NBUF, u)

                @pl.when(t > 0)
                def _():
                    pltpu.make_async_copy(stag[v], out_slice(2 * v),
                                          osems[v]).wait()

                pltpu.async_copy(stag[v], out_slice(t * NBUF + 2 * v),
                                 osems[v])
            return 0

        lax.fori_loop(0, NITER, body, 0)
        for v in range(2):
            pltpu.make_async_copy(stag[v], out_slice(2 * v), osems[v]).wait()

    return k(gT, idx3d)


def _bn_body(yT_ref, gb_ref, out_ref, acc_ref):
    i = pl.program_id(0)

    @pl.when(i == 0)
    def _():
        acc_ref[...] = jnp.zeros_like(acc_ref)

    @pl.when(i < NBLK_C)
    def _():
        rows = lax.broadcasted_iota(jnp.int32, (BF_C, O), 0) + i * BF_C
        y = jnp.where(rows < F, yT_ref[...].astype(jnp.float32), 0.0)
        acc_ref[0:1, :] += jnp.sum(y, axis=0, keepdims=True)
        acc_ref[1:2, :] += jnp.sum(y * y, axis=0, keepdims=True)

    @pl.when(i >= NBLK_C)
    def _():
        mean = acc_ref[0:1, :] / F
        var = acc_ref[1:2, :] / F - mean * mean
        inv = lax.rsqrt(var + 1e-5)
        scale = gb_ref[0:1, :] * inv
        shift = gb_ref[1:2, :] - mean * scale
        y = yT_ref[...].astype(jnp.float32)
        z = jnp.maximum(y * scale + shift, 0.0)
        eye = (lax.broadcasted_iota(jnp.int32, (O, O), 0)
               == lax.broadcasted_iota(jnp.int32, (O, O), 1)
               ).astype(jnp.float32)
        out_ref[...] = lax.dot_general(
            eye, z,
            dimension_numbers=(((1,), (1,)), ((), ())),
            preferred_element_type=jnp.float32,
        )


def _bn_norm(yT, gb):
    return pl.pallas_call(
        _bn_body,
        grid=(2 * NBLK_C,),
        in_specs=[
            pl.BlockSpec((BF_C, O), lambda i: (i % NBLK_C, 0)),
            pl.BlockSpec((2, O), lambda i: (0, 0)),
        ],
        out_specs=pl.BlockSpec((O, BF_C),
                               lambda i: (0, jnp.maximum(i - NBLK_C, 0))),
        out_shape=jax.ShapeDtypeStruct((O, F), jnp.float32),
        scratch_shapes=[pltpu.VMEM((2, O), jnp.float32)],
    )(yT, gb)


def kernel(fea, ring_n, pool_idx, W, b, gamma, beta):
    del b
    fea2d = fea[0]
    gT = _matmul_transposed(fea2d, W)

    idx = jnp.concatenate([pool_idx[:, None], ring_n[0]], axis=1)
    idx = jnp.pad(idx, ((0, F_PAD - F), (0, 0)), constant_values=F_FULL)
    idx3d = idx.reshape(NW, GATHERS_PER_W, ROWS_PER_GATHER)

    yT = _sc_gather_sum(gT, idx3d)
    gb = jnp.stack([gamma, beta])
    out2d = _bn_norm(yT, gb)
    return out2d[None]


# speedup vs baseline: 2.9833x; 1.0606x over previous
import functools

import jax
import jax.numpy as jnp
from jax import lax
from jax.experimental import pallas as pl
from jax.experimental.pallas import tpu as pltpu
from jax.experimental.pallas import tpu_sc as plsc

C = 128
O = 128
F_FULL = 50000
F = 25000
K = 16
NK = K + 1

BF_A = 1024
F_FULL_PAD = 49 * BF_A

NW = 32
FACES_PER_W = 784
F_PAD = NW * FACES_PER_W
FACES_PER_GATHER = 4
ROWS_PER_GATHER = FACES_PER_GATHER * NK
GATHERS_PER_W = FACES_PER_W // FACES_PER_GATHER
NBUF = 4
NITER = GATHERS_PER_W // NBUF

BF_C = 896
NBLK_C = F_PAD // BF_C


def _matmul_body(fea_ref, w_ref, out_ref):
    i = pl.program_id(0)
    y = lax.dot_general(
        fea_ref[...], w_ref[...],
        dimension_numbers=(((0,), (1,)), ((), ())),
        preferred_element_type=jnp.float32,
    )
    rows = lax.broadcasted_iota(jnp.int32, (BF_A, O), 0) + i * BF_A
    out_ref[...] = jnp.where(rows < F_FULL, y, 0.0).astype(jnp.bfloat16)


def _matmul_transposed(fea2d, W):
    return pl.pallas_call(
        _matmul_body,
        grid=(F_FULL_PAD // BF_A,),
        in_specs=[
            pl.BlockSpec((C, BF_A), lambda i: (0, i)),
            pl.BlockSpec((O, C), lambda i: (0, 0)),
        ],
        out_specs=pl.BlockSpec((BF_A, O), lambda i: (i, 0)),
        out_shape=jax.ShapeDtypeStruct((F_FULL_PAD, O), jnp.bfloat16),
    )(fea2d, W)


def _sc_gather_sum(gT, idx3d):
    mesh = plsc.VectorSubcoreMesh(core_axis_name="c", subcore_axis_name="s")

    @functools.partial(
        pl.kernel,
        mesh=mesh,
        out_type=jax.ShapeDtypeStruct((F_PAD, O), jnp.bfloat16),
        compiler_params=pltpu.CompilerParams(use_tc_tiling_on_sc=False),
        scratch_types=[
            pltpu.VMEM((GATHERS_PER_W, ROWS_PER_GATHER), jnp.int32),
            *[pltpu.VMEM((ROWS_PER_GATHER, O), jnp.bfloat16)
              for _ in range(NBUF)],
            *[pltpu.VMEM((2 * FACES_PER_GATHER, O), jnp.bfloat16)
              for _ in range(2)],
            *[pltpu.SemaphoreType.DMA for _ in range(NBUF + 2)],
        ],
    )
    def k(gT_hbm, idx_hbm, out_hbm, idx_v, b0, b1, b2, b3,
          st0, st1, s0, s1, s2, s3, t0, t1):
        bufs = (b0, b1, b2, b3)
        stag = (st0, st1)
        sems = (s0, s1, s2, s3)
        osems = (t0, t1)
        wid = lax.axis_index("s") * 2 + lax.axis_index("c")
        pltpu.sync_copy(idx_hbm.at[wid], idx_v)
        row_base = wid * FACES_PER_W

        def issue(g, u):
            return pltpu.async_copy(gT_hbm.at[idx_v.at[g]], bufs[u], sems[u])

        def out_slice(g):
            return out_hbm.at[pl.ds(row_base + g * FACES_PER_GATHER,
                                    2 * FACES_PER_GATHER)]

        for u in range(NBUF):
            issue(u, u)

        def body(t, _):
            for v in range(2):
                for h in range(2):
                    u = 2 * v + h
                    g = t * NBUF + u
                    pltpu.make_async_copy(gT_hbm.at[idx_v.at[g]],
                                          bufs[u], sems[u]).wait()
                    for j in range(FACES_PER_GATHER):
                        for c in range(O // 32):
                            sl = pl.ds(c * 32, 32)
                            acc = bufs[u][j * NK, sl]
                            for r in range(1, NK):
                                acc = acc + bufs[u][j * NK + r, sl]
                            stag[v][h * FACES_PER_GATHER + j, sl] = acc

                    @pl.when(t < NITER - 1)
                    def _():
                        issue(g + NBUF, u)

                @pl.when(t > 0)
                def _():
                    pltpu.make_async_copy(stag[v], out_slice(2 * v),
                                          osems[v]).wait()

                pltpu.async_copy(stag[v], out_slice(t * NBUF + 2 * v),
                                 osems[v])
            return 0

        lax.fori_loop(0, NITER, body, 0)
        for v in range(2):
            pltpu.make_async_copy(stag[v], out_slice(2 * v), osems[v]).wait()

    return k(gT, idx3d)


def _bn_body(yT_ref, gb_ref, out_ref, acc_ref):
    i = pl.program_id(0)

    @pl.when(i == 0)
    def _():
        acc_ref[...] = jnp.zeros_like(acc_ref)

    @pl.when(i < NBLK_C)
    def _():
        rows = lax.broadcasted_iota(jnp.int32, (BF_C, O), 0) + i * BF_C
        y = jnp.where(rows < F, yT_ref[...].astype(jnp.float32), 0.0)
        acc_ref[0:1, :] += jnp.sum(y, axis=0, keepdims=True)
        acc_ref[1:2, :] += jnp.sum(y * y, axis=0, keepdims=True)

    @pl.when(i >= NBLK_C)
    def _():
        mean = acc_ref[0:1, :] / F
        var = acc_ref[1:2, :] / F - mean * mean
        inv = lax.rsqrt(var + 1e-5)
        scale = gb_ref[0:1, :] * inv
        shift = gb_ref[1:2, :] - mean * scale
        y = yT_ref[...].astype(jnp.float32)
        z = jnp.maximum(y * scale + shift, 0.0)
        eye = (lax.broadcasted_iota(jnp.int32, (O, O), 0)
               == lax.broadcasted_iota(jnp.int32, (O, O), 1)
               ).astype(jnp.float32)
        out_ref[...] = lax.dot_general(
            eye, z,
            dimension_numbers=(((1,), (1,)), ((), ())),
            preferred_element_type=jnp.float32,
        )


def _bn_norm(yT, gb):
    return pl.pallas_call(
        _bn_body,
        grid=(2 * NBLK_C,),
        in_specs=[
            pl.BlockSpec((BF_C, O), lambda i: (i % NBLK_C, 0)),
            pl.BlockSpec((2, O), lambda i: (0, 0)),
        ],
        out_specs=pl.BlockSpec((O, BF_C),
                               lambda i: (0, jnp.maximum(i - NBLK_C, 0))),
        out_shape=jax.ShapeDtypeStruct((O, F), jnp.float32),
        scratch_shapes=[pltpu.VMEM((2, O), jnp.float32)],
    )(yT, gb)


def kernel(fea, ring_n, pool_idx, W, b, gamma, beta):
    del b
    fea2d = fea[0]
    gT = _matmul_transposed(fea2d, W)

    idx = jnp.concatenate([pool_idx[:, None], ring_n[0]], axis=1)
    idx = jnp.pad(idx, ((0, F_PAD - F), (0, 0)), constant_values=F_FULL)
    idx3d = idx.reshape(NW, GATHERS_PER_W, ROWS_PER_GATHER)

    yT = _sc_gather_sum(gT, idx3d)
    gb = jnp.stack([gamma, beta])
    out2d = _bn_norm(yT, gb)
    return out2d[None]
